# baseline (speedup 1.0000x reference)
import functools

import jax
import jax.numpy as jnp
from jax import lax
from jax.experimental import pallas as pl
from jax.experimental.pallas import tpu as pltpu
from jax.experimental.pallas import tpu_sc as plsc

BATCH = 1024
SEQ = 50
NUM_ROWS = BATCH * SEQ
DIM = 128

_info = plsc.get_sparse_core_info()
NUM_CORES = _info.num_cores
NUM_SUBCORES = _info.num_subcores
NUM_WORKERS = NUM_CORES * NUM_SUBCORES

ROWS_PER_WORKER = BATCH // NUM_WORKERS
PER_WORKER = ROWS_PER_WORKER * SEQ
G = 4
CHUNK = G * SEQ
NUM_CHUNKS = ROWS_PER_WORKER // G
NBUF = 4

_mesh = plsc.VectorSubcoreMesh(core_axis_name="c", subcore_axis_name="s")

_scratch = (
    [pltpu.VMEM((PER_WORKER,), jnp.int32)]
    + [pltpu.VMEM((CHUNK, DIM), jnp.float32) for _ in range(NBUF)]
    + [pltpu.SemaphoreType.DMA for _ in range(NBUF)]
)


@functools.partial(
    pl.kernel,
    mesh=_mesh,
    out_type=jax.ShapeDtypeStruct((BATCH, SEQ, DIM), jnp.float32),
    scratch_types=_scratch,
    compiler_params=pltpu.CompilerParams(use_tc_tiling_on_sc=True),
)
def _gather(idx_hbm, table_hbm, out_hbm, idx_v, *bufs_and_sems):
    rows = bufs_and_sems[:NBUF]
    sems = bufs_and_sems[NBUF:]
    wid = lax.axis_index("s") * NUM_CORES + lax.axis_index("c")
    base = wid * PER_WORKER
    row0 = wid * ROWS_PER_WORKER

    pltpu.sync_copy(idx_hbm.at[pl.ds(base, PER_WORKER)], idx_v)

    def start_gather(c, b):
        return pltpu.async_copy(
            table_hbm.at[idx_v.at[pl.ds(c * CHUNK, CHUNK)]], rows[b], sems[b]
        )

    def start_stores(c, b):
        return [
            pltpu.async_copy(
                rows[b].at[pl.ds(m * SEQ, SEQ)],
                out_hbm.at[row0 + c * G + m],
                sems[b],
            )
            for m in range(G)
        ]

    gathers = [None] * NUM_CHUNKS
    stores = [None] * NUM_CHUNKS
    for c in range(NBUF):
        gathers[c] = start_gather(c, c)
    for c in range(NUM_CHUNKS):
        b = c % NBUF
        gathers[c].wait()
        stores[c] = start_stores(c, b)
        nc = c + NBUF
        if nc < NUM_CHUNKS:
            for s in stores[c]:
                s.wait()
            gathers[nc] = start_gather(nc, b)
    for c in range(NUM_CHUNKS - NBUF, NUM_CHUNKS):
        for s in stores[c]:
            s.wait()


def kernel(symbol_indices, table):
    idx = symbol_indices.reshape(-1).astype(jnp.int32)
    return _gather(idx, table)


# speedup vs baseline: 1.5738x; 1.5738x over previous
import functools

import jax
import jax.numpy as jnp
from jax import lax
from jax.experimental import pallas as pl
from jax.experimental.pallas import tpu as pltpu
from jax.experimental.pallas import tpu_sc as plsc

BATCH = 1024
SEQ = 50
NUM_ROWS = BATCH * SEQ
DIM = 128

_info = plsc.get_sparse_core_info()
NUM_CORES = _info.num_cores
NUM_SUBCORES = _info.num_subcores
NUM_WORKERS = NUM_CORES * NUM_SUBCORES

PER_WORKER = NUM_ROWS // NUM_WORKERS
CHUNK = 80
NUM_CHUNKS = PER_WORKER // CHUNK
NBUF = 8

_mesh = plsc.VectorSubcoreMesh(core_axis_name="c", subcore_axis_name="s")

_scratch = (
    [pltpu.VMEM((PER_WORKER,), jnp.int32)]
    + [pltpu.VMEM((CHUNK, DIM), jnp.float32) for _ in range(NBUF)]
    + [pltpu.SemaphoreType.DMA for _ in range(NBUF)]
)


@functools.partial(
    pl.kernel,
    mesh=_mesh,
    out_type=jax.ShapeDtypeStruct((NUM_ROWS, DIM), jnp.float32),
    scratch_types=_scratch,
)
def _gather(idx_hbm, table_hbm, out_hbm, idx_v, *bufs_and_sems):
    rows = bufs_and_sems[:NBUF]
    sems = bufs_and_sems[NBUF:]
    wid = lax.axis_index("s") * NUM_CORES + lax.axis_index("c")
    base = wid * PER_WORKER

    pltpu.sync_copy(idx_hbm.at[pl.ds(base, PER_WORKER)], idx_v)

    gathers = [None] * NUM_CHUNKS
    scatters = [None] * NUM_CHUNKS
    for c in range(NBUF):
        gathers[c] = pltpu.async_copy(
            table_hbm.at[idx_v.at[pl.ds(c * CHUNK, CHUNK)]], rows[c], sems[c]
        )
    for c in range(NUM_CHUNKS):
        b = c % NBUF
        gathers[c].wait()
        scatters[c] = pltpu.async_copy(
            rows[b], out_hbm.at[pl.ds(base + c * CHUNK, CHUNK)], sems[b]
        )
        nc = c + NBUF
        if nc < NUM_CHUNKS:
            scatters[c].wait()
            gathers[nc] = pltpu.async_copy(
                table_hbm.at[idx_v.at[pl.ds(nc * CHUNK, CHUNK)]], rows[b], sems[b]
            )
    for c in range(NUM_CHUNKS - NBUF, NUM_CHUNKS):
        scatters[c].wait()


def kernel(symbol_indices, table):
    idx_t = symbol_indices.astype(jnp.int32).T.reshape(-1)
    out = _gather(idx_t, table)
    return out.reshape(SEQ, BATCH, DIM).transpose(1, 0, 2)


# speedup vs baseline: 1.6014x; 1.0175x over previous
import functools

import jax
import jax.numpy as jnp
from jax import lax
from jax.experimental import pallas as pl
from jax.experimental.pallas import tpu as pltpu
from jax.experimental.pallas import tpu_sc as plsc

BATCH = 1024
SEQ = 50
NUM_ROWS = BATCH * SEQ
DIM = 128

_info = plsc.get_sparse_core_info()
NUM_CORES = _info.num_cores
NUM_SUBCORES = _info.num_subcores
NUM_WORKERS = NUM_CORES * NUM_SUBCORES

PER_WORKER = NUM_ROWS // NUM_WORKERS
CHUNK = 200
NUM_CHUNKS = PER_WORKER // CHUNK
NBUF = 4

_mesh = plsc.VectorSubcoreMesh(core_axis_name="c", subcore_axis_name="s")

_scratch = (
    [pltpu.VMEM((PER_WORKER,), jnp.int32)]
    + [pltpu.VMEM((CHUNK, DIM), jnp.float32) for _ in range(NBUF)]
    + [pltpu.SemaphoreType.DMA for _ in range(NBUF)]
)


@functools.partial(
    pl.kernel,
    mesh=_mesh,
    out_type=jax.ShapeDtypeStruct((NUM_ROWS, DIM), jnp.float32),
    scratch_types=_scratch,
)
def _gather(idx_hbm, table_hbm, out_hbm, idx_v, *bufs_and_sems):
    rows = bufs_and_sems[:NBUF]
    sems = bufs_and_sems[NBUF:]
    wid = lax.axis_index("s") * NUM_CORES + lax.axis_index("c")
    base = wid * PER_WORKER

    pltpu.sync_copy(idx_hbm.at[pl.ds(base, PER_WORKER)], idx_v)

    gathers = [None] * NUM_CHUNKS
    scatters = [None] * NUM_CHUNKS
    for c in range(NBUF):
        gathers[c] = pltpu.async_copy(
            table_hbm.at[idx_v.at[pl.ds(c * CHUNK, CHUNK)]], rows[c], sems[c]
        )
    for c in range(NUM_CHUNKS):
        b = c % NBUF
        gathers[c].wait()
        scatters[c] = pltpu.async_copy(
            rows[b], out_hbm.at[pl.ds(base + c * CHUNK, CHUNK)], sems[b]
        )
        nc = c + NBUF
        if nc < NUM_CHUNKS:
            scatters[c].wait()
            gathers[nc] = pltpu.async_copy(
                table_hbm.at[idx_v.at[pl.ds(nc * CHUNK, CHUNK)]], rows[b], sems[b]
            )
    for c in range(NUM_CHUNKS - NBUF, NUM_CHUNKS):
        scatters[c].wait()


def kernel(symbol_indices, table):
    idx_t = symbol_indices.astype(jnp.int32).T.reshape(-1)
    out = _gather(idx_t, table)
    return out.reshape(SEQ, BATCH, DIM).transpose(1, 0, 2)
